# baseline (speedup 1.0000x reference)
import functools

import jax
import jax.numpy as jnp
from jax import lax
from jax.experimental import pallas as pl
from jax.experimental.pallas import tpu as pltpu
from jax.experimental.pallas import tpu_sc as plsc

N = 10000
E = 320000
D_IN = 128
D_OUT = 128
H = 64
NC = 2
NS = 16
CHUNK = 128
CC = 112
NPAD = 10112
PAD_SPREAD = 48

A_CHUNKS = 80
A_PAD = 32 * A_CHUNKS * CHUNK - E
C_CHUNKS = 180
C_PAD = NS * C_CHUNKS * CC - E

ROWS_PER_TILE = NPAD // NS

_MESH = plsc.VectorSubcoreMesh(core_axis_name="c", subcore_axis_name="s")


def _zero_fill(buf, nrows, width):
    zv = jnp.zeros((16,), jnp.float32)

    def body(r, _):
        for j in range(width // 16):
            buf[r, pl.ds(j * 16, 16)] = zv
        return 0

    lax.fori_loop(0, nrows, body, 0)


def _zero_accum_slice(zbuf, bs, accum, base, nrows):
    full = nrows // bs
    for k in range(full):
        pltpu.sync_copy(zbuf, accum.at[pl.ds(base + k * bs, bs)])
    rem = nrows - full * bs
    if rem:
        pltpu.sync_copy(zbuf.at[pl.ds(0, rem)],
                        accum.at[pl.ds(base + full * bs, rem)])


@functools.partial(
    pl.kernel,
    mesh=_MESH,
    out_type=jax.ShapeDtypeStruct((NC * NPAD, 16), jnp.float32),
    scratch_types=[
        pltpu.VMEM_SHARED((NPAD, 16), jnp.float32),
        pltpu.VMEM((CHUNK, 16), jnp.float32),
        pltpu.VMEM((CHUNK, 16), jnp.float32),
        pltpu.VMEM((A_CHUNKS, CHUNK), jnp.int32),
        pltpu.SemaphoreType.DMA,
        pltpu.SemaphoreType.DMA,
    ],
    compiler_params=pltpu.CompilerParams(use_tc_tiling_on_sc=False),
)
def _deg_kernel(dsta_hbm, ones_hbm, degp_hbm,
                accum, ones_v, zero_v, idx_v, sem0, sem1):
    c = lax.axis_index("c")
    s = lax.axis_index("s")
    w = s * NC + c
    sems = (sem0, sem1)

    _zero_fill(zero_v, CHUNK, 16)
    _zero_accum_slice(zero_v, CHUNK, accum, s * ROWS_PER_TILE, ROWS_PER_TILE)
    pltpu.sync_copy(ones_hbm, ones_v)
    pltpu.sync_copy(dsta_hbm.at[pl.ds(w * A_CHUNKS, A_CHUNKS)], idx_v)
    plsc.subcore_barrier()

    for j in range(2):
        pltpu.async_copy(ones_v, accum.at[idx_v.at[j]], sems[j], add=True)

    def body(i, _):
        for p in range(2):
            j = 2 * i + p
            pltpu.make_async_copy(
                ones_v, accum.at[idx_v.at[j - 2]], sems[p]).wait()
            pltpu.async_copy(ones_v, accum.at[idx_v.at[j]], sems[p], add=True)
        return 0

    lax.fori_loop(1, A_CHUNKS // 2, body, 0)
    for p in range(2):
        pltpu.make_async_copy(
            ones_v, accum.at[idx_v.at[A_CHUNKS - 2 + p]], sems[p]).wait()
    plsc.subcore_barrier()

    lo = s * ROWS_PER_TILE
    pltpu.sync_copy(accum.at[pl.ds(lo, ROWS_PER_TILE)],
                    degp_hbm.at[pl.ds(c * NPAD + lo, ROWS_PER_TILE)])


_BLK = 2000


def _matmul_body(x_ref, w_ref, xw_ref):
    xw_ref[...] = jnp.dot(x_ref[...], w_ref[...],
                          preferred_element_type=jnp.float32)


def _matmul(x, w):
    grid = (N // _BLK,)
    return pl.pallas_call(
        _matmul_body,
        grid=grid,
        in_specs=[
            pl.BlockSpec((_BLK, D_IN), lambda i: (i, 0)),
            pl.BlockSpec((D_IN, D_OUT), lambda i: (0, 0)),
        ],
        out_specs=pl.BlockSpec((_BLK, D_OUT), lambda i: (i, 0)),
        out_shape=jax.ShapeDtypeStruct((N, D_OUT), jnp.float32),
    )(x, w)


def _scale_body(xw_ref, d_ref, y_ref, rdeg_ref):
    deg = d_ref[0, :, 0:1] + d_ref[1, :, 0:1]
    rdeg = lax.rsqrt(jnp.maximum(deg, 1.0))
    y = xw_ref[...] * rdeg
    y_ref[0, :, :] = y[:, :H]
    y_ref[1, :, :] = y[:, H:]
    rdeg_ref[...] = jnp.broadcast_to(rdeg, (rdeg.shape[0], 16))


def _scale(xw, degp):
    grid = (N // _BLK,)
    return pl.pallas_call(
        _scale_body,
        grid=grid,
        in_specs=[
            pl.BlockSpec((_BLK, D_OUT), lambda i: (i, 0)),
            pl.BlockSpec((2, _BLK, 16), lambda i: (0, i, 0)),
        ],
        out_specs=[
            pl.BlockSpec((2, _BLK, H), lambda i: (0, i, 0)),
            pl.BlockSpec((_BLK, 16), lambda i: (i, 0)),
        ],
        out_shape=[
            jax.ShapeDtypeStruct((2, N, H), jnp.float32),
            jax.ShapeDtypeStruct((NPAD, 16), jnp.float32),
        ],
    )(xw, degp)


_NB = 3
_G = C_CHUNKS // _NB


@functools.partial(
    pl.kernel,
    mesh=_MESH,
    out_type=jax.ShapeDtypeStruct((NC * NPAD, H), jnp.float32),
    scratch_types=(
        [
            pltpu.VMEM_SHARED((NPAD, H), jnp.float32),
            pltpu.VMEM((C_CHUNKS, CC), jnp.int32),
            pltpu.VMEM((C_CHUNKS, CC), jnp.int32),
        ]
        + [pltpu.VMEM((CC, H), jnp.float32)] * (2 * _NB)
        + [pltpu.SemaphoreType.DMA] * (4 * _NB)
        + [pltpu.VMEM((CC, 16), jnp.float32)]
    ),
    compiler_params=pltpu.CompilerParams(use_tc_tiling_on_sc=False),
)
def _agg_kernel(y_hbm, srcs_hbm, dsts_hbm, rdeg_hbm, out_hbm,
                accum, src_i, dst_i, *bufs):
    c = lax.axis_index("c")
    s = lax.axis_index("s")
    rows = (bufs[0:_NB], bufs[_NB:2 * _NB])
    gsem = (bufs[2 * _NB:3 * _NB], bufs[3 * _NB:4 * _NB])
    ssem = (bufs[4 * _NB:5 * _NB], bufs[5 * _NB:6 * _NB])

    _zero_fill(rows[0][0], CC, H)
    _zero_accum_slice(rows[0][0], CC, accum, s * ROWS_PER_TILE, ROWS_PER_TILE)
    pltpu.sync_copy(
        srcs_hbm.at[pl.ds((c * NS + s) * C_CHUNKS, C_CHUNKS)], src_i)
    pltpu.sync_copy(dsts_hbm.at[pl.ds(s * C_CHUNKS, C_CHUNKS)], dst_i)
    plsc.subcore_barrier()

    def gather(j, p, b):
        pltpu.async_copy(y_hbm.at[src_i.at[j]], rows[p][b], gsem[p][b])

    def gather_wait(j, p, b):
        pltpu.make_async_copy(y_hbm.at[src_i.at[j]], rows[p][b],
                              gsem[p][b]).wait()

    def scatter(j, p, b):
        pltpu.async_copy(rows[p][b], accum.at[dst_i.at[j]], ssem[p][b],
                         add=True)

    def scatter_wait(j, p, b):
        pltpu.make_async_copy(rows[p][b], accum.at[dst_i.at[j]],
                              ssem[p][b]).wait()

    for g0 in range(2):
        for b in range(_NB):
            gather(g0 * _NB + b, g0, b)
        for b in range(_NB):
            gather_wait(g0 * _NB + b, g0, b)
            scatter(g0 * _NB + b, g0, b)

    def body(i, _):
        for p in range(2):
            g = 2 * i + p
            j0 = g * _NB
            for b in range(_NB):
                scatter_wait(j0 + b - 2 * _NB, p, b)
                gather(j0 + b, p, b)
            for b in range(_NB):
                gather_wait(j0 + b, p, b)
                scatter(j0 + b, p, b)
        return 0

    lax.fori_loop(1, _G // 2, body, 0)
    for p in range(2):
        for b in range(_NB):
            scatter_wait((_G - 2 + p) * _NB + b, p, b)

    rdeg_v = bufs[6 * _NB]
    lo = s * ROWS_PER_TILE
    plsc.subcore_barrier()

    buf = rows[0][0]
    nblk = -(-ROWS_PER_TILE // CC)
    for k in range(nblk):
        bs = min(CC, ROWS_PER_TILE - k * CC)
        pltpu.sync_copy(rdeg_hbm.at[pl.ds(lo + k * CC, bs), :],
                        rdeg_v.at[pl.ds(0, bs)])
        pltpu.sync_copy(accum.at[pl.ds(lo + k * CC, bs)],
                        buf.at[pl.ds(0, bs)])

        def scale_row(r, _):
            rv = rdeg_v[r, :]
            for j in range(H // 16):
                sl = pl.ds(j * 16, 16)
                buf[r, sl] = buf[r, sl] * rv
            return 0

        lax.fori_loop(0, bs, scale_row, 0)
        pltpu.sync_copy(buf.at[pl.ds(0, bs)],
                        out_hbm.at[pl.ds(c * NPAD + lo + k * CC, bs)])


def kernel(X, edge_index, weight):
    src = edge_index[0]
    dst = edge_index[1]

    pad_a = N + (jnp.arange(A_PAD, dtype=jnp.int32) % PAD_SPREAD)
    dsta = jnp.concatenate([dst, pad_a]).reshape(32 * A_CHUNKS, CHUNK)

    pad_src = (jnp.arange(C_PAD, dtype=jnp.int32) * 977) % N
    pad_dst = N + (jnp.arange(C_PAD, dtype=jnp.int32) % PAD_SPREAD)
    srcc = jnp.concatenate([src, pad_src])
    srcs = jnp.concatenate([srcc, srcc + N]).reshape(2 * NS * C_CHUNKS, CC)
    dsts = jnp.concatenate([dst, pad_dst]).reshape(NS * C_CHUNKS, CC)

    ones = jnp.ones((CHUNK, 16), jnp.float32)

    xw = _matmul(X, weight)
    degp = _deg_kernel(dsta, ones).reshape(2, NPAD, 16)
    y, rdeg = _scale(xw, degp)
    o = _agg_kernel(y.reshape(2 * N, H), srcs, dsts, rdeg)
    return jnp.concatenate([o[:N], o[NPAD:NPAD + N]], axis=1)


# speedup vs baseline: 1.0487x; 1.0487x over previous
import functools

import jax
import jax.numpy as jnp
from jax import lax
from jax.experimental import pallas as pl
from jax.experimental.pallas import tpu as pltpu
from jax.experimental.pallas import tpu_sc as plsc

N = 10000
E = 320000
D_IN = 128
D_OUT = 128
H = 64
NC = 2
NS = 16
CHUNK = 128
CC = 112
NPAD = 10112
PAD_SPREAD = 48

A_CHUNKS = 80
A_PAD = 32 * A_CHUNKS * CHUNK - E
C_CHUNKS = 180
C_PAD = NS * C_CHUNKS * CC - E

ROWS_PER_TILE = NPAD // NS

_MESH = plsc.VectorSubcoreMesh(core_axis_name="c", subcore_axis_name="s")


def _zero_fill(buf, nrows, width):
    zv = jnp.zeros((16,), jnp.float32)

    def body(r, _):
        for j in range(width // 16):
            buf[r, pl.ds(j * 16, 16)] = zv
        return 0

    lax.fori_loop(0, nrows, body, 0)


def _zero_accum_slice(zbuf, bs, accum, base, nrows):
    full = nrows // bs
    for k in range(full):
        pltpu.sync_copy(zbuf, accum.at[pl.ds(base + k * bs, bs)])
    rem = nrows - full * bs
    if rem:
        pltpu.sync_copy(zbuf.at[pl.ds(0, rem)],
                        accum.at[pl.ds(base + full * bs, rem)])


@functools.partial(
    pl.kernel,
    mesh=_MESH,
    out_type=jax.ShapeDtypeStruct((NC * NPAD, 16), jnp.float32),
    scratch_types=[
        pltpu.VMEM_SHARED((NPAD, 16), jnp.float32),
        pltpu.VMEM((CHUNK, 16), jnp.float32),
        pltpu.VMEM((CHUNK, 16), jnp.float32),
        pltpu.VMEM((A_CHUNKS, CHUNK), jnp.int32),
        pltpu.SemaphoreType.DMA,
        pltpu.SemaphoreType.DMA,
    ],
    compiler_params=pltpu.CompilerParams(use_tc_tiling_on_sc=False),
)
def _deg_kernel(dsta_hbm, ones_hbm, degp_hbm,
                accum, ones_v, zero_v, idx_v, sem0, sem1):
    c = lax.axis_index("c")
    s = lax.axis_index("s")
    w = s * NC + c
    sems = (sem0, sem1)

    _zero_fill(zero_v, CHUNK, 16)
    _zero_accum_slice(zero_v, CHUNK, accum, s * ROWS_PER_TILE, ROWS_PER_TILE)
    pltpu.sync_copy(ones_hbm, ones_v)
    pltpu.sync_copy(dsta_hbm.at[pl.ds(w * A_CHUNKS, A_CHUNKS)], idx_v)
    plsc.subcore_barrier()

    for j in range(2):
        pltpu.async_copy(ones_v, accum.at[idx_v.at[j]], sems[j], add=True)

    def body(i, _):
        for p in range(2):
            j = 2 * i + p
            pltpu.make_async_copy(
                ones_v, accum.at[idx_v.at[j - 2]], sems[p]).wait()
            pltpu.async_copy(ones_v, accum.at[idx_v.at[j]], sems[p], add=True)
        return 0

    lax.fori_loop(1, A_CHUNKS // 2, body, 0)
    for p in range(2):
        pltpu.make_async_copy(
            ones_v, accum.at[idx_v.at[A_CHUNKS - 2 + p]], sems[p]).wait()
    plsc.subcore_barrier()

    lo = s * ROWS_PER_TILE
    pltpu.sync_copy(accum.at[pl.ds(lo, ROWS_PER_TILE)],
                    degp_hbm.at[pl.ds(c * NPAD + lo, ROWS_PER_TILE)])


_BLK = 2000


def _matmul_body(x_ref, w_ref, xw_ref):
    xw_ref[...] = jnp.dot(x_ref[...], w_ref[...],
                          preferred_element_type=jnp.float32)


def _matmul(x, w):
    grid = (N // _BLK,)
    return pl.pallas_call(
        _matmul_body,
        grid=grid,
        in_specs=[
            pl.BlockSpec((_BLK, D_IN), lambda i: (i, 0)),
            pl.BlockSpec((D_IN, D_OUT), lambda i: (0, 0)),
        ],
        out_specs=pl.BlockSpec((_BLK, D_OUT), lambda i: (i, 0)),
        out_shape=jax.ShapeDtypeStruct((N, D_OUT), jnp.float32),
    )(x, w)


def _scale_body(xw_ref, d_ref, y_ref):
    deg = d_ref[0, :, 0:1] + d_ref[1, :, 0:1]
    rdeg = lax.rsqrt(jnp.maximum(deg, 1.0))
    y = xw_ref[...] * rdeg
    y_ref[0, :, :] = y[:, :H]
    y_ref[1, :, :] = y[:, H:]


def _scale(xw, degp):
    grid = (N // _BLK,)
    return pl.pallas_call(
        _scale_body,
        grid=grid,
        in_specs=[
            pl.BlockSpec((_BLK, D_OUT), lambda i: (i, 0)),
            pl.BlockSpec((2, _BLK, 16), lambda i: (0, i, 0)),
        ],
        out_specs=pl.BlockSpec((2, _BLK, H), lambda i: (0, i, 0)),
        out_shape=jax.ShapeDtypeStruct((2, N, H), jnp.float32),
    )(xw, degp)


_NB = 3
_G = C_CHUNKS // _NB


@functools.partial(
    pl.kernel,
    mesh=_MESH,
    out_type=jax.ShapeDtypeStruct((NC * NPAD, H), jnp.float32),
    scratch_types=(
        [
            pltpu.VMEM_SHARED((NPAD, H), jnp.float32),
            pltpu.VMEM((C_CHUNKS, CC), jnp.int32),
            pltpu.VMEM((C_CHUNKS, CC), jnp.int32),
        ]
        + [pltpu.VMEM((CC, H), jnp.float32)] * (2 * _NB)
        + [pltpu.SemaphoreType.DMA] * (4 * _NB)
    ),
    compiler_params=pltpu.CompilerParams(use_tc_tiling_on_sc=False),
)
def _agg_kernel(y_hbm, srcs_hbm, dsts_hbm, out_hbm,
                accum, src_i, dst_i, *bufs):
    c = lax.axis_index("c")
    s = lax.axis_index("s")
    rows = (bufs[0:_NB], bufs[_NB:2 * _NB])
    gsem = (bufs[2 * _NB:3 * _NB], bufs[3 * _NB:4 * _NB])
    ssem = (bufs[4 * _NB:5 * _NB], bufs[5 * _NB:6 * _NB])

    _zero_fill(rows[0][0], CC, H)
    _zero_accum_slice(rows[0][0], CC, accum, s * ROWS_PER_TILE, ROWS_PER_TILE)
    pltpu.sync_copy(
        srcs_hbm.at[pl.ds((c * NS + s) * C_CHUNKS, C_CHUNKS)], src_i)
    pltpu.sync_copy(dsts_hbm.at[pl.ds(s * C_CHUNKS, C_CHUNKS)], dst_i)
    plsc.subcore_barrier()

    def gather(j, p, b):
        pltpu.async_copy(y_hbm.at[src_i.at[j]], rows[p][b], gsem[p][b])

    def gather_wait(j, p, b):
        pltpu.make_async_copy(y_hbm.at[src_i.at[j]], rows[p][b],
                              gsem[p][b]).wait()

    def scatter(j, p, b):
        pltpu.async_copy(rows[p][b], accum.at[dst_i.at[j]], ssem[p][b],
                         add=True)

    def scatter_wait(j, p, b):
        pltpu.make_async_copy(rows[p][b], accum.at[dst_i.at[j]],
                              ssem[p][b]).wait()

    for g0 in range(2):
        for b in range(_NB):
            gather(g0 * _NB + b, g0, b)
        for b in range(_NB):
            gather_wait(g0 * _NB + b, g0, b)
            scatter(g0 * _NB + b, g0, b)

    def body(i, _):
        for p in range(2):
            g = 2 * i + p
            j0 = g * _NB
            for b in range(_NB):
                scatter_wait(j0 + b - 2 * _NB, p, b)
                gather(j0 + b, p, b)
            for b in range(_NB):
                gather_wait(j0 + b, p, b)
                scatter(j0 + b, p, b)
        return 0

    lax.fori_loop(1, _G // 2, body, 0)
    for p in range(2):
        for b in range(_NB):
            scatter_wait((_G - 2 + p) * _NB + b, p, b)

    plsc.subcore_barrier()

    lo = s * ROWS_PER_TILE
    pltpu.sync_copy(accum.at[pl.ds(lo, ROWS_PER_TILE)],
                    out_hbm.at[pl.ds(c * NPAD + lo, ROWS_PER_TILE)])


def _finalize_body(o_ref, d_ref, out_ref):
    deg = d_ref[0, :, 0:1] + d_ref[1, :, 0:1]
    rdeg = lax.rsqrt(jnp.maximum(deg, 1.0))
    out_ref[...] = jnp.concatenate(
        [o_ref[0, :, :] * rdeg, o_ref[1, :, :] * rdeg], axis=1)


def _finalize(o, degp):
    grid = (N // _BLK,)
    return pl.pallas_call(
        _finalize_body,
        grid=grid,
        in_specs=[
            pl.BlockSpec((2, _BLK, H), lambda i: (0, i, 0)),
            pl.BlockSpec((2, _BLK, 16), lambda i: (0, i, 0)),
        ],
        out_specs=pl.BlockSpec((_BLK, D_OUT), lambda i: (i, 0)),
        out_shape=jax.ShapeDtypeStruct((N, D_OUT), jnp.float32),
    )(o, degp)


def kernel(X, edge_index, weight):
    src = edge_index[0]
    dst = edge_index[1]

    pad_a = N + (jnp.arange(A_PAD, dtype=jnp.int32) % PAD_SPREAD)
    dsta = jnp.concatenate([dst, pad_a]).reshape(32 * A_CHUNKS, CHUNK)

    pad_src = (jnp.arange(C_PAD, dtype=jnp.int32) * 977) % N
    pad_dst = N + (jnp.arange(C_PAD, dtype=jnp.int32) % PAD_SPREAD)
    srcc = jnp.concatenate([src, pad_src])
    srcs = jnp.concatenate([srcc, srcc + N]).reshape(2 * NS * C_CHUNKS, CC)
    dsts = jnp.concatenate([dst, pad_dst]).reshape(NS * C_CHUNKS, CC)

    ones = jnp.ones((CHUNK, 16), jnp.float32)

    xw = _matmul(X, weight)
    degp = _deg_kernel(dsta, ones).reshape(2, NPAD, 16)
    y = _scale(xw, degp)
    o = _agg_kernel(y.reshape(2 * N, H), srcs, dsts).reshape(2, NPAD, H)
    return _finalize(o, degp)
